# baseline (speedup 1.0000x reference)
import functools

import jax
import jax.numpy as jnp
from jax import lax
from jax.experimental import pallas as pl
from jax.experimental.pallas import tpu as pltpu
from jax.experimental.pallas import tpu_sc as plsc

NC = 2
NS = 16
NW = NC * NS
LANES = 16

VOCAB = 1000000
EMBED_DIM = 128
BATCH = 1024
SEQ_LEN = 200

ROWS = BATCH * SEQ_LEN
ROWS_PER_W = ROWS // NW
CHUNK = 40
GPB = SEQ_LEN // CHUNK
N_CHUNKS = ROWS_PER_W // CHUNK
N_BLOCKS = ROWS_PER_W // SEQ_LEN


def _emb_kernel(table_hbm, idx_hbm, pos_hbm, out_hbm, pos_v, idx_v, buf_v, sem):
    wid = lax.axis_index("s") * NC + lax.axis_index("c")

    pltpu.sync_copy(idx_hbm.at[wid], idx_v)
    pltpu.sync_copy(pos_hbm, pos_v)

    def block_body(j, _):
        copies = [
            pltpu.async_copy(
                table_hbm.at[idx_v.at[j * GPB + k]],
                buf_v.at[pl.ds(k * CHUNK, CHUNK)],
                sem,
            )
            for k in range(GPB)
        ]
        for c in copies:
            c.wait()

        def row_body(r, _):
            for c in range(EMBED_DIM // LANES):
                sl = pl.ds(c * LANES, LANES)
                buf_v[r, sl] = buf_v[r, sl] + pos_v[r, sl]
            return 0

        lax.fori_loop(0, SEQ_LEN, row_body, 0)

        start = wid * ROWS_PER_W + j * SEQ_LEN
        pltpu.sync_copy(buf_v, out_hbm.at[pl.ds(start, SEQ_LEN)])
        return 0

    lax.fori_loop(0, N_BLOCKS, block_body, 0)


@jax.jit
def _run(word_emb, idx3, pos_emb):
    mesh = plsc.VectorSubcoreMesh(
        core_axis_name="c", subcore_axis_name="s",
        num_cores=NC, num_subcores=NS,
    )
    f = functools.partial(
        pl.kernel,
        out_type=jax.ShapeDtypeStruct((ROWS, EMBED_DIM), jnp.float32),
        mesh=mesh,
        scratch_types=[
            pltpu.VMEM((SEQ_LEN, EMBED_DIM), jnp.float32),
            pltpu.VMEM((N_CHUNKS, CHUNK), jnp.int32),
            pltpu.VMEM((SEQ_LEN, EMBED_DIM), jnp.float32),
            pltpu.SemaphoreType.DMA,
        ],
    )(_emb_kernel)
    return f(word_emb, idx3, pos_emb)


def kernel(input_ids, word_emb, pos_emb):
    idx3 = input_ids.reshape(NW, N_CHUNKS, CHUNK).astype(jnp.int32)
    out = _run(word_emb, idx3, pos_emb)
    return out.reshape(BATCH, SEQ_LEN, EMBED_DIM)


# speedup vs baseline: 1.5057x; 1.5057x over previous
import functools

import jax
import jax.numpy as jnp
from jax import lax
from jax.experimental import pallas as pl
from jax.experimental.pallas import tpu as pltpu
from jax.experimental.pallas import tpu_sc as plsc

NC = 2
NS = 16
NW = NC * NS
LANES = 16

VOCAB = 1000000
EMBED_DIM = 128
BATCH = 1024
SEQ_LEN = 200

ROWS = BATCH * SEQ_LEN
ROWS_PER_W = ROWS // NW
CHUNK = 40
GPB = SEQ_LEN // CHUNK
N_CHUNKS = ROWS_PER_W // CHUNK
N_BLOCKS = ROWS_PER_W // SEQ_LEN
HALF_ITERS = N_BLOCKS // 2


def _emb_kernel(table_hbm, idx_hbm, pos_hbm, out_hbm,
                pos_v, idx_v, buf0, buf1, g0, g1, s0, s1):
    wid = lax.axis_index("s") * NC + lax.axis_index("c")
    base = wid * ROWS_PER_W

    pltpu.sync_copy(idx_hbm.at[wid], idx_v)
    pltpu.sync_copy(pos_hbm, pos_v)

    def fire(j, buf, gsem):
        for k in range(GPB):
            pltpu.async_copy(
                table_hbm.at[idx_v.at[j * GPB + k]],
                buf.at[pl.ds(k * CHUNK, CHUNK)],
                gsem,
            )

    def wait_g(j, buf, gsem):
        for k in range(GPB):
            pltpu.make_async_copy(
                table_hbm.at[idx_v.at[j * GPB + k]],
                buf.at[pl.ds(k * CHUNK, CHUNK)],
                gsem,
            ).wait()

    def add_pos(buf):
        def row_body(r, _):
            for c in range(EMBED_DIM // LANES):
                sl = pl.ds(c * LANES, LANES)
                buf[r, sl] = buf[r, sl] + pos_v[r, sl]
            return 0
        lax.fori_loop(0, SEQ_LEN, row_body, 0)

    def fire_store(j, buf, ssem):
        pltpu.async_copy(buf, out_hbm.at[pl.ds(base + j * SEQ_LEN, SEQ_LEN)], ssem)

    def wait_s(j, buf, ssem):
        pltpu.make_async_copy(
            buf, out_hbm.at[pl.ds(base + j * SEQ_LEN, SEQ_LEN)], ssem
        ).wait()

    fire(0, buf0, g0)

    def body(j2, _):
        jA = 2 * j2
        jB = jA + 1

        wait_g(jA, buf0, g0)

        @pl.when(j2 > 0)
        def _():
            wait_s(jA - 1, buf1, s1)

        fire(jB, buf1, g1)
        add_pos(buf0)
        fire_store(jA, buf0, s0)

        wait_g(jB, buf1, g1)
        wait_s(jA, buf0, s0)

        @pl.when(j2 < HALF_ITERS - 1)
        def _():
            fire(jA + 2, buf0, g0)

        add_pos(buf1)
        fire_store(jB, buf1, s1)
        return 0

    lax.fori_loop(0, HALF_ITERS, body, 0)
    wait_s(N_BLOCKS - 1, buf1, s1)


@jax.jit
def _run(word_emb, idx3, pos_emb):
    mesh = plsc.VectorSubcoreMesh(
        core_axis_name="c", subcore_axis_name="s",
        num_cores=NC, num_subcores=NS,
    )
    f = functools.partial(
        pl.kernel,
        out_type=jax.ShapeDtypeStruct((ROWS, EMBED_DIM), jnp.float32),
        mesh=mesh,
        scratch_types=[
            pltpu.VMEM((SEQ_LEN, EMBED_DIM), jnp.float32),
            pltpu.VMEM((N_CHUNKS, CHUNK), jnp.int32),
            pltpu.VMEM((SEQ_LEN, EMBED_DIM), jnp.float32),
            pltpu.VMEM((SEQ_LEN, EMBED_DIM), jnp.float32),
            pltpu.SemaphoreType.DMA,
            pltpu.SemaphoreType.DMA,
            pltpu.SemaphoreType.DMA,
            pltpu.SemaphoreType.DMA,
        ],
    )(_emb_kernel)
    return f(word_emb, idx3, pos_emb)


def kernel(input_ids, word_emb, pos_emb):
    idx3 = input_ids.reshape(NW, N_CHUNKS, CHUNK).astype(jnp.int32)
    out = _run(word_emb, idx3, pos_emb)
    return out.reshape(BATCH, SEQ_LEN, EMBED_DIM)


# speedup vs baseline: 1.7840x; 1.1848x over previous
import functools

import jax
import jax.numpy as jnp
from jax import lax
from jax.experimental import pallas as pl
from jax.experimental.pallas import tpu as pltpu
from jax.experimental.pallas import tpu_sc as plsc

NC = 2
NS = 16
NW = NC * NS
LANES = 16

VOCAB = 1000000
EMBED_DIM = 128
BATCH = 1024
SEQ_LEN = 200

ROWS = BATCH * SEQ_LEN
ROWS_PER_W = ROWS // NW
CHUNK = 100
GPB = SEQ_LEN // CHUNK
N_CHUNKS = ROWS_PER_W // CHUNK
N_BLOCKS = ROWS_PER_W // SEQ_LEN
HALF_ITERS = N_BLOCKS // 2
VPR = EMBED_DIM // LANES


def _emb_kernel(table_hbm, idx_hbm, pos_hbm, out_hbm,
                pos_v, idx_v, buf0, buf1, g0, g1, s0, s1):
    wid = lax.axis_index("s") * NC + lax.axis_index("c")
    base = wid * ROWS_PER_W

    pltpu.sync_copy(idx_hbm.at[wid], idx_v)
    pltpu.sync_copy(pos_hbm, pos_v)

    def init_pos(buf):
        def row_body(r, _):
            for c in range(VPR):
                sl = pl.ds(c * LANES, LANES)
                buf[r, sl] = pos_v[r, sl]
            return 0
        lax.fori_loop(0, SEQ_LEN, row_body, 0)

    def fire(j, buf, gsem):
        for k in range(GPB):
            pltpu.async_copy(
                table_hbm.at[idx_v.at[j * GPB + k]],
                buf.at[pl.ds(k * CHUNK, CHUNK)],
                gsem,
                add=True,
            )

    def wait_g(j, buf, gsem):
        for k in range(GPB):
            pltpu.make_async_copy(
                table_hbm.at[idx_v.at[j * GPB + k]],
                buf.at[pl.ds(k * CHUNK, CHUNK)],
                gsem,
            ).wait()

    def fire_store(j, buf, ssem):
        pltpu.async_copy(buf, out_hbm.at[pl.ds(base + j * SEQ_LEN, SEQ_LEN)], ssem)

    def wait_s(j, buf, ssem):
        pltpu.make_async_copy(
            buf, out_hbm.at[pl.ds(base + j * SEQ_LEN, SEQ_LEN)], ssem
        ).wait()

    init_pos(buf0)
    fire(0, buf0, g0)
    init_pos(buf1)
    fire(1, buf1, g1)

    def body(j2, _):
        jA = 2 * j2
        jB = jA + 1

        wait_g(jA, buf0, g0)
        fire_store(jA, buf0, s0)
        wait_s(jA, buf0, s0)

        @pl.when(j2 < HALF_ITERS - 1)
        def _():
            init_pos(buf0)
            fire(jA + 2, buf0, g0)

        wait_g(jB, buf1, g1)
        fire_store(jB, buf1, s1)
        wait_s(jB, buf1, s1)

        @pl.when(j2 < HALF_ITERS - 1)
        def _():
            init_pos(buf1)
            fire(jB + 2, buf1, g1)

        return 0

    lax.fori_loop(0, HALF_ITERS, body, 0)


@jax.jit
def _run(word_emb, idx3, pos_emb):
    mesh = plsc.VectorSubcoreMesh(
        core_axis_name="c", subcore_axis_name="s",
        num_cores=NC, num_subcores=NS,
    )
    f = functools.partial(
        pl.kernel,
        out_type=jax.ShapeDtypeStruct((ROWS, EMBED_DIM), jnp.float32),
        mesh=mesh,
        scratch_types=[
            pltpu.VMEM((SEQ_LEN, EMBED_DIM), jnp.float32),
            pltpu.VMEM((N_CHUNKS, CHUNK), jnp.int32),
            pltpu.VMEM((SEQ_LEN, EMBED_DIM), jnp.float32),
            pltpu.VMEM((SEQ_LEN, EMBED_DIM), jnp.float32),
            pltpu.SemaphoreType.DMA,
            pltpu.SemaphoreType.DMA,
            pltpu.SemaphoreType.DMA,
            pltpu.SemaphoreType.DMA,
        ],
    )(_emb_kernel)
    return f(word_emb, idx3, pos_emb)


def kernel(input_ids, word_emb, pos_emb):
    idx3 = input_ids.reshape(NW, N_CHUNKS, CHUNK).astype(jnp.int32)
    out = _run(word_emb, idx3, pos_emb)
    return out.reshape(BATCH, SEQ_LEN, EMBED_DIM)


# speedup vs baseline: 1.7873x; 1.0019x over previous
import functools

import jax
import jax.numpy as jnp
from jax import lax
from jax.experimental import pallas as pl
from jax.experimental.pallas import tpu as pltpu
from jax.experimental.pallas import tpu_sc as plsc

NC = 2
NS = 16
NW = NC * NS
LANES = 16

VOCAB = 1000000
EMBED_DIM = 128
BATCH = 1024
SEQ_LEN = 200

ROWS = BATCH * SEQ_LEN
ROWS_PER_W = ROWS // NW
CHUNK = 100
GPB = SEQ_LEN // CHUNK
N_CHUNKS = ROWS_PER_W // CHUNK
N_BLOCKS = ROWS_PER_W // SEQ_LEN
HALF_ITERS = N_BLOCKS // 2
VPR = EMBED_DIM // LANES


def _emb_kernel(table_hbm, idx_hbm, pos_hbm, out_hbm,
                pos_v, idx_v, buf0, buf1, g0, g1, s0, s1):
    wid = lax.axis_index("s") * NC + lax.axis_index("c")
    base = wid * ROWS_PER_W

    pltpu.sync_copy(idx_hbm.at[wid], idx_v)
    pltpu.sync_copy(pos_hbm, pos_v)

    def init_pos(buf):
        def row_body(r, _):
            for c in range(VPR):
                sl = pl.ds(c * LANES, LANES)
                buf[r, sl] = pos_v[r, sl]
            return 0
        lax.fori_loop(0, SEQ_LEN, row_body, 0)

    def fire(j, buf, gsem):
        for k in range(GPB):
            pltpu.async_copy(
                table_hbm.at[idx_v.at[j * GPB + k]],
                buf.at[pl.ds(k * CHUNK, CHUNK)],
                gsem,
                add=True,
            )

    def wait_g(j, buf, gsem):
        for k in range(GPB):
            pltpu.make_async_copy(
                table_hbm.at[idx_v.at[j * GPB + k]],
                buf.at[pl.ds(k * CHUNK, CHUNK)],
                gsem,
            ).wait()

    def fire_store(j, buf, ssem):
        pltpu.async_copy(buf, out_hbm.at[pl.ds(base + j * SEQ_LEN, SEQ_LEN)], ssem)

    def wait_s(j, buf, ssem):
        pltpu.make_async_copy(
            buf, out_hbm.at[pl.ds(base + j * SEQ_LEN, SEQ_LEN)], ssem
        ).wait()

    init_pos(buf0)
    fire(0, buf0, g0)
    init_pos(buf1)
    fire(1, buf1, g1)

    def body(j2, _):
        jA = 2 * j2
        jB = jA + 1

        wait_g(jA, buf0, g0)
        fire_store(jA, buf0, s0)
        wait_s(jA, buf0, s0)

        @pl.when(j2 < HALF_ITERS - 1)
        def _():
            init_pos(buf0)
            fire(jA + 2, buf0, g0)

        wait_g(jB, buf1, g1)
        fire_store(jB, buf1, s1)
        wait_s(jB, buf1, s1)

        @pl.when(j2 < HALF_ITERS - 1)
        def _():
            init_pos(buf1)
            fire(jB + 2, buf1, g1)

        return 0

    lax.fori_loop(0, HALF_ITERS, body, 0)


@jax.jit
def _run(input_ids, word_emb, pos_emb):
    idx3 = input_ids.reshape(NW, N_CHUNKS, CHUNK).astype(jnp.int32)
    mesh = plsc.VectorSubcoreMesh(
        core_axis_name="c", subcore_axis_name="s",
        num_cores=NC, num_subcores=NS,
    )
    f = functools.partial(
        pl.kernel,
        out_type=jax.ShapeDtypeStruct((ROWS, EMBED_DIM), jnp.float32),
        mesh=mesh,
        scratch_types=[
            pltpu.VMEM((SEQ_LEN, EMBED_DIM), jnp.float32),
            pltpu.VMEM((N_CHUNKS, CHUNK), jnp.int32),
            pltpu.VMEM((SEQ_LEN, EMBED_DIM), jnp.float32),
            pltpu.VMEM((SEQ_LEN, EMBED_DIM), jnp.float32),
            pltpu.SemaphoreType.DMA,
            pltpu.SemaphoreType.DMA,
            pltpu.SemaphoreType.DMA,
            pltpu.SemaphoreType.DMA,
        ],
    )(_emb_kernel)
    out = f(word_emb, idx3, pos_emb)
    return out.reshape(BATCH, SEQ_LEN, EMBED_DIM)


def kernel(input_ids, word_emb, pos_emb):
    return _run(input_ids, word_emb, pos_emb)
